# baseline (speedup 1.0000x reference)
import functools

import jax
import jax.numpy as jnp
from jax import lax
from jax.experimental import pallas as pl
from jax.experimental.pallas import tpu as pltpu
from jax.experimental.pallas import tpu_sc as plsc

NC = 2
NS = 16
NW = NC * NS
LANES = 16
CH = 128

def _mesh():
    return plsc.VectorSubcoreMesh(core_axis_name="c", subcore_axis_name="s")


def _sc_gather2(x, row, col):
    n, h = x.shape
    e = row.shape[0]
    assert e % NW == 0
    epw = e // NW
    n_full, rem = divmod(epw, CH)
    assert epw % 8 == 0 and rem % 8 == 0

    assert n_full % 2 == 0
    npair = n_full // 2

    out_t = jax.ShapeDtypeStruct((e, h), x.dtype)
    scratch = [
        pltpu.VMEM((CH,), jnp.int32), pltpu.VMEM((CH,), jnp.int32),
        pltpu.VMEM((CH,), jnp.int32), pltpu.VMEM((CH,), jnp.int32),
        pltpu.VMEM((CH, h), x.dtype), pltpu.VMEM((CH, h), x.dtype),
        pltpu.VMEM((CH, h), x.dtype), pltpu.VMEM((CH, h), x.dtype),
        pltpu.SemaphoreType.DMA, pltpu.SemaphoreType.DMA,
        pltpu.SemaphoreType.DMA, pltpu.SemaphoreType.DMA,
        pltpu.SemaphoreType.DMA, pltpu.SemaphoreType.DMA,
    ]
    if rem:
        scratch += [
            pltpu.VMEM((rem,), jnp.int32), pltpu.VMEM((rem,), jnp.int32),
            pltpu.VMEM((rem, h), x.dtype), pltpu.VMEM((rem, h), x.dtype),
        ]

    @functools.partial(pl.kernel, out_type=(out_t, out_t), mesh=_mesh(),
                       scratch_types=scratch,
                       compiler_params=pltpu.CompilerParams(
                           use_tc_tiling_on_sc=False))
    def k(x_hbm, row_hbm, col_hbm, src_hbm, dst_hbm,
          idxr0, idxr1, idxc0, idxc1, bufr0, bufr1, bufc0, bufc1,
          semi0, semi1, semg0, semg1, semo0, semo1, *tail):
        idxr, idxc = [idxr0, idxr1], [idxc0, idxc1]
        bufr, bufc = [bufr0, bufr1], [bufc0, bufc1]
        semi, semg, semo = [semi0, semi1], [semg0, semg1], [semo0, semo1]
        wid = lax.axis_index("s") * NC + lax.axis_index("c")
        base_w = wid * epw

        def idx_cp(p, b):
            return (pltpu.make_async_copy(row_hbm.at[pl.ds(b, CH)],
                                          idxr[p], semi[p]),
                    pltpu.make_async_copy(col_hbm.at[pl.ds(b, CH)],
                                          idxc[p], semi[p]))

        def gat_cp(p):
            return (pltpu.make_async_copy(x_hbm.at[idxr[p]], bufr[p], semg[p]),
                    pltpu.make_async_copy(x_hbm.at[idxc[p]], bufc[p], semg[p]))

        def out_cp(p, b):
            return (pltpu.make_async_copy(bufr[p], src_hbm.at[pl.ds(b, CH)],
                                          semo[p]),
                    pltpu.make_async_copy(bufc[p], dst_hbm.at[pl.ds(b, CH)],
                                          semo[p]))

        def start2(c):
            c[0].start()
            c[1].start()

        def wait2(c):
            c[0].wait()
            c[1].wait()

        start2(idx_cp(0, base_w))
        start2(idx_cp(1, base_w + CH))
        wait2(idx_cp(0, base_w))
        start2(gat_cp(0))
        wait2(idx_cp(1, base_w + CH))
        start2(gat_cp(1))

        @pl.loop(0, npair)
        def _(j):
            c0 = base_w + (2 * j) * CH
            c1 = c0 + CH
            wait2(gat_cp(0))
            start2(out_cp(0, c0))
            wait2(gat_cp(1))
            start2(out_cp(1, c1))

            @pl.when(j < npair - 1)
            def _():
                start2(idx_cp(0, c0 + 2 * CH))
                start2(idx_cp(1, c1 + 2 * CH))
                wait2(idx_cp(0, c0 + 2 * CH))
                wait2(out_cp(0, c0))
                start2(gat_cp(0))
                wait2(idx_cp(1, c1 + 2 * CH))
                wait2(out_cp(1, c1))
                start2(gat_cp(1))

        blast = base_w + (n_full - 2) * CH
        wait2(out_cp(0, blast))
        wait2(out_cp(1, blast + CH))

        if rem:
            idxrt, idxct, bufrt, bufct = tail
            bt = base_w + n_full * CH
            pltpu.sync_copy(row_hbm.at[pl.ds(bt, rem)], idxrt)
            pltpu.sync_copy(col_hbm.at[pl.ds(bt, rem)], idxct)
            pltpu.sync_copy(x_hbm.at[idxrt], bufrt)
            pltpu.sync_copy(x_hbm.at[idxct], bufct)
            pltpu.sync_copy(bufrt, src_hbm.at[pl.ds(bt, rem)])
            pltpu.sync_copy(bufct, dst_hbm.at[pl.ds(bt, rem)])

    return k(x, row, col)


def _sc_segsum(vals, col, zeros, count_mode=False):
    e, w = vals.shape
    n = zeros.shape[0]
    assert e % NC == 0
    epc = e // NC
    assert epc % NS == 0
    eps = epc // NS
    n_full, rem = divmod(eps, CH)
    assert eps % 8 == 0 and rem % 8 == 0
    rps = (n // NS) & ~7
    rps_last = n - (NS - 1) * rps
    assert rps % 8 == 0 and rps_last % 8 == 0 and rps_last >= rps

    assert n_full % 2 == 0
    npair = n_full // 2

    out_t = jax.ShapeDtypeStruct((n, w), vals.dtype)
    scratch = [
        pltpu.VMEM_SHARED((n, w), vals.dtype),
        pltpu.VMEM((CH,), jnp.int32), pltpu.VMEM((CH,), jnp.int32),
        pltpu.VMEM((CH, w), vals.dtype), pltpu.VMEM((CH, w), vals.dtype),
        pltpu.SemaphoreType.DMA, pltpu.SemaphoreType.DMA,
        pltpu.SemaphoreType.DMA, pltpu.SemaphoreType.DMA,
    ]
    if rem:
        scratch += [pltpu.VMEM((rem,), jnp.int32),
                    pltpu.VMEM((rem, w), vals.dtype)]

    @functools.partial(pl.kernel, out_type=(out_t, out_t), mesh=_mesh(),
                       scratch_types=scratch)
    def k(vals_hbm, col_hbm, zeros_hbm, out0_hbm, out1_hbm, acc_sh,
          idx0, idx1, buf0, buf1, semiv0, semiv1, semsc0, semsc1, *tail):
        idxv, bufv = [idx0, idx1], [buf0, buf1]
        semiv, semsc = [semiv0, semiv1], [semsc0, semsc1]
        core = lax.axis_index("c")
        sid = lax.axis_index("s")

        if count_mode:
            @pl.loop(0, CH)
            def _(r):
                @pl.loop(0, w // LANES)
                def _(cc):
                    buf0[r, pl.ds(cc * LANES, LANES)] = jnp.ones(
                        (LANES,), vals.dtype)

        r0 = sid * rps

        @pl.when(sid < NS - 1)
        def _():
            pltpu.sync_copy(zeros_hbm.at[pl.ds(r0, rps)],
                            acc_sh.at[pl.ds(r0, rps)])

        @pl.when(sid == NS - 1)
        def _():
            pltpu.sync_copy(zeros_hbm.at[pl.ds(r0, rps_last)],
                            acc_sh.at[pl.ds(r0, rps_last)])

        plsc.subcore_barrier()

        base = core * epc + sid * eps

        def iv_cp(p, b):
            cs = [pltpu.make_async_copy(col_hbm.at[pl.ds(b, CH)],
                                        idxv[p], semiv[p])]
            if not count_mode:
                cs.append(pltpu.make_async_copy(vals_hbm.at[pl.ds(b, CH)],
                                                bufv[p], semiv[p]))
            return cs

        def sc_src(p):
            return bufv[0] if count_mode else bufv[p]

        def startall(cs):
            for c in cs:
                c.start()

        def waitall(cs):
            for c in cs:
                c.wait()

        def sc_start(p):
            pltpu.async_copy(sc_src(p), acc_sh.at[idxv[p]], semsc[p],
                             add=True)

        def sc_wait(p):
            pltpu.make_async_copy(sc_src(p), acc_sh.at[idxv[p]],
                                  semsc[p]).wait()

        startall(iv_cp(0, base))
        startall(iv_cp(1, base + CH))

        @pl.loop(0, npair)
        def _(j):
            c0 = base + (2 * j) * CH
            c1 = c0 + CH
            waitall(iv_cp(0, c0))
            sc_start(0)
            waitall(iv_cp(1, c1))
            sc_start(1)

            @pl.when(j < npair - 1)
            def _():
                sc_wait(0)
                startall(iv_cp(0, c0 + 2 * CH))
                sc_wait(1)
                startall(iv_cp(1, c1 + 2 * CH))

        sc_wait(0)
        sc_wait(1)

        if rem:
            idx_t, buf_t = tail
            bt = base + n_full * CH
            pltpu.sync_copy(col_hbm.at[pl.ds(bt, rem)], idx_t)
            if count_mode:
                @pl.loop(0, rem)
                def _(r):
                    @pl.loop(0, w // LANES)
                    def _(cc):
                        buf_t[r, pl.ds(cc * LANES, LANES)] = jnp.ones(
                            (LANES,), vals.dtype)
            else:
                pltpu.sync_copy(vals_hbm.at[pl.ds(bt, rem)], buf_t)
            pltpu.sync_copy(buf_t, acc_sh.at[idx_t], add=True)

        plsc.subcore_barrier()

        for cc, out_hbm in ((0, out0_hbm), (1, out1_hbm)):
            @pl.when((core == cc) & (sid < NS - 1))
            def _():
                pltpu.sync_copy(acc_sh.at[pl.ds(r0, rps)],
                                out_hbm.at[pl.ds(r0, rps)])

            @pl.when((core == cc) & (sid == NS - 1))
            def _():
                pltpu.sync_copy(acc_sh.at[pl.ds(r0, rps_last)],
                                out_hbm.at[pl.ds(r0, rps_last)])

    return k(vals, col, zeros)


def _dot(a, b):
    return jax.lax.dot_general(a.astype(jnp.bfloat16), b.astype(jnp.bfloat16),
                               (((1,), (0,)), ((), ())),
                               preferred_element_type=jnp.float32)


def _tc_edge(src, dst, ea, pe, pn1):
    e, h = ea.shape
    be = 2560
    assert e % be == 0
    grid = (e // be,)

    w1s, w1d, w1e = pe["W1"][:h], pe["W1"][h:2 * h], pe["W1"][2 * h:]
    v1s, v1e = pn1["W1"][:h], pn1["W1"][h:]
    b1 = pe["b1"].reshape(1, h)
    b2 = pe["b2"].reshape(1, h)
    c2 = pn1["b2"].reshape(1, h)
    z = jnp.zeros((h, h), jnp.float32)

    def perm(w):
        return jnp.concatenate([w[0::2], w[1::2]], axis=0)

    wa = jnp.concatenate([
        perm(jnp.concatenate([w1s, v1s], axis=1)),
        perm(jnp.concatenate([w1d, z], axis=1)),
        jnp.concatenate([w1e, z], axis=1)], axis=0)
    w2v = jnp.matmul(pe["W2"], v1e, precision=jax.lax.Precision.HIGHEST)
    wb = jnp.concatenate([pe["W2"], w2v], axis=1)
    c1p = (pn1["b1"]
           + jnp.matmul(b2, v1e,
                        precision=jax.lax.Precision.HIGHEST)).reshape(1, h)

    row_spec = pl.BlockSpec((be, h), lambda i: (i, 0))
    packed_spec = pl.BlockSpec((be, h // 2), lambda i: (i, 0))
    wa_spec = pl.BlockSpec((3 * h, 2 * h), lambda i: (0, 0))
    wb_spec = pl.BlockSpec((h, 2 * h), lambda i: (0, 0))
    w_spec = pl.BlockSpec((h, h), lambda i: (0, 0))
    b_spec = pl.BlockSpec((1, h), lambda i: (0, 0))

    def body(src_r, dst_r, ea_r, wa_r, b1_r, wb_r, b2_r, c1p_r, v2_r, c2_r,
             e_out, m_out):
        def unpack(ref):
            v = ref[...]
            lo = jax.lax.bitcast_convert_type(v << 16, jnp.float32)
            hi = jax.lax.bitcast_convert_type(v & jnp.int32(-65536),
                                              jnp.float32)
            return jnp.concatenate([lo, hi], axis=1)

        cat = jnp.concatenate([unpack(src_r), unpack(dst_r),
                               ea_r[...]], axis=1)
        a = _dot(cat, wa_r[...])
        hh = jnp.maximum(a[:, :h] + b1_r[...], 0.0)
        gs = a[:, h:]
        bb = _dot(hh, wb_r[...])
        e2 = bb[:, :h] + b2_r[...]
        e_out[...] = e2
        g = jnp.maximum(gs + bb[:, h:] + c1p_r[...], 0.0)
        m_out[...] = _dot(g, v2_r[...]) + c2_r[...]

    sds = jax.ShapeDtypeStruct((e, h), jnp.float32)
    return pl.pallas_call(
        body,
        grid=grid,
        in_specs=[packed_spec, packed_spec, row_spec,
                  wa_spec, b_spec, wb_spec, b_spec, b_spec, w_spec, b_spec],
        out_specs=[row_spec, row_spec],
        out_shape=(sds, sds),
        compiler_params=pltpu.CompilerParams(
            dimension_semantics=("parallel",)),
    )(src, dst, ea, wa, b1, wb, b2, c1p, pn1["W2"], c2)


def _tc_node(x, s0, s1, c0, c1, pn2):
    n, h = x.shape
    bn = 1000
    assert n % bn == 0
    grid = (n // bn,)

    b1 = pn2["b1"].reshape(1, h)
    b2 = pn2["b2"].reshape(1, h)

    row_spec = pl.BlockSpec((bn, h), lambda i: (i, 0))
    cnt_spec = pl.BlockSpec((bn, h), lambda i: (i, 0))
    w1_spec = pl.BlockSpec((2 * h, h), lambda i: (0, 0))
    w_spec = pl.BlockSpec((h, h), lambda i: (0, 0))
    b_spec = pl.BlockSpec((1, h), lambda i: (0, 0))

    def body(x_r, s0_r, s1_r, c0_r, c1_r, w1_r, b1_r, w2_r, b2_r, out_r):
        cnt = jnp.maximum(c0_r[:, 0:1] + c1_r[:, 0:1], 1.0)
        agg = (s0_r[...] + s1_r[...]) / cnt
        cat = jnp.concatenate([x_r[...], agg], axis=1)
        hh = jnp.maximum(_dot(cat, w1_r[...]) + b1_r[...], 0.0)
        out_r[...] = _dot(hh, w2_r[...]) + b2_r[...]

    return pl.pallas_call(
        body,
        grid=grid,
        in_specs=[row_spec, row_spec, row_spec, cnt_spec, cnt_spec,
                  w1_spec, b_spec, w_spec, b_spec],
        out_specs=pl.BlockSpec((bn, h), lambda i: (i, 0)),
        out_shape=jax.ShapeDtypeStruct((n, h), jnp.float32),
        compiler_params=pltpu.CompilerParams(
            dimension_semantics=("parallel",)),
    )(x, s0, s1, c0, c1, pn2["W1"], b1, pn2["W2"], b2)


def kernel(x, edge_index, edge_attr, params):
    n, h = x.shape
    e = edge_attr.shape[0]
    row = edge_index[0]
    col = edge_index[1]

    zeros_s = jnp.zeros((n, h), jnp.float32)
    c0, c1 = _sc_segsum(edge_attr, col, zeros_s, count_mode=True)

    for p in params:
        xi = jax.lax.bitcast_convert_type(
            x.astype(jnp.bfloat16).reshape(n, h // 2, 2), jnp.int32)
        src, dst = _sc_gather2(xi, row, col)
        e_new, m = _tc_edge(src, dst, edge_attr, p["edge"], p["node1"])
        s0, s1 = _sc_segsum(m, col, zeros_s)
        x = _tc_node(x, s0, s1, c0, c1, p["node2"])
        edge_attr = e_new
    return (x, edge_attr)


# speedup vs baseline: 1.2302x; 1.2302x over previous
import functools

import jax
import jax.numpy as jnp
from jax import lax
from jax.experimental import pallas as pl
from jax.experimental.pallas import tpu as pltpu
from jax.experimental.pallas import tpu_sc as plsc

NC = 2
NS = 16
NW = NC * NS
LANES = 16
CH = 128

def _mesh():
    return plsc.VectorSubcoreMesh(core_axis_name="c", subcore_axis_name="s")


def _sc_gather2(x, row, col):
    n, h = x.shape
    e = row.shape[0]
    assert e % NW == 0
    epw = e // NW
    n_full, rem = divmod(epw, CH)
    assert epw % 8 == 0 and rem % 8 == 0

    assert n_full % 2 == 0
    npair = n_full // 2

    out_t = jax.ShapeDtypeStruct((e, h), x.dtype)
    scratch = [
        pltpu.VMEM((CH,), jnp.int32), pltpu.VMEM((CH,), jnp.int32),
        pltpu.VMEM((CH,), jnp.int32), pltpu.VMEM((CH,), jnp.int32),
        pltpu.VMEM((CH, h), x.dtype), pltpu.VMEM((CH, h), x.dtype),
        pltpu.VMEM((CH, h), x.dtype), pltpu.VMEM((CH, h), x.dtype),
        pltpu.SemaphoreType.DMA, pltpu.SemaphoreType.DMA,
        pltpu.SemaphoreType.DMA, pltpu.SemaphoreType.DMA,
        pltpu.SemaphoreType.DMA, pltpu.SemaphoreType.DMA,
    ]
    if rem:
        scratch += [
            pltpu.VMEM((rem,), jnp.int32), pltpu.VMEM((rem,), jnp.int32),
            pltpu.VMEM((rem, h), x.dtype), pltpu.VMEM((rem, h), x.dtype),
        ]

    @functools.partial(pl.kernel, out_type=(out_t, out_t), mesh=_mesh(),
                       scratch_types=scratch)
    def k(x_hbm, row_hbm, col_hbm, src_hbm, dst_hbm,
          idxr0, idxr1, idxc0, idxc1, bufr0, bufr1, bufc0, bufc1,
          semi0, semi1, semg0, semg1, semo0, semo1, *tail):
        idxr, idxc = [idxr0, idxr1], [idxc0, idxc1]
        bufr, bufc = [bufr0, bufr1], [bufc0, bufc1]
        semi, semg, semo = [semi0, semi1], [semg0, semg1], [semo0, semo1]
        wid = lax.axis_index("s") * NC + lax.axis_index("c")
        base_w = wid * epw

        def idx_cp(p, b):
            return (pltpu.make_async_copy(row_hbm.at[pl.ds(b, CH)],
                                          idxr[p], semi[p]),
                    pltpu.make_async_copy(col_hbm.at[pl.ds(b, CH)],
                                          idxc[p], semi[p]))

        def gat_cp(p):
            return (pltpu.make_async_copy(x_hbm.at[idxr[p]], bufr[p], semg[p]),
                    pltpu.make_async_copy(x_hbm.at[idxc[p]], bufc[p], semg[p]))

        def out_cp(p, b):
            return (pltpu.make_async_copy(bufr[p], src_hbm.at[pl.ds(b, CH)],
                                          semo[p]),
                    pltpu.make_async_copy(bufc[p], dst_hbm.at[pl.ds(b, CH)],
                                          semo[p]))

        def start2(c):
            c[0].start()
            c[1].start()

        def wait2(c):
            c[0].wait()
            c[1].wait()

        start2(idx_cp(0, base_w))
        start2(idx_cp(1, base_w + CH))
        wait2(idx_cp(0, base_w))
        start2(gat_cp(0))
        wait2(idx_cp(1, base_w + CH))
        start2(gat_cp(1))

        @pl.loop(0, npair)
        def _(j):
            c0 = base_w + (2 * j) * CH
            c1 = c0 + CH
            wait2(gat_cp(0))
            start2(out_cp(0, c0))
            wait2(gat_cp(1))
            start2(out_cp(1, c1))

            @pl.when(j < npair - 1)
            def _():
                start2(idx_cp(0, c0 + 2 * CH))
                start2(idx_cp(1, c1 + 2 * CH))
                wait2(idx_cp(0, c0 + 2 * CH))
                wait2(out_cp(0, c0))
                start2(gat_cp(0))
                wait2(idx_cp(1, c1 + 2 * CH))
                wait2(out_cp(1, c1))
                start2(gat_cp(1))

        blast = base_w + (n_full - 2) * CH
        wait2(out_cp(0, blast))
        wait2(out_cp(1, blast + CH))

        if rem:
            idxrt, idxct, bufrt, bufct = tail
            bt = base_w + n_full * CH
            pltpu.sync_copy(row_hbm.at[pl.ds(bt, rem)], idxrt)
            pltpu.sync_copy(col_hbm.at[pl.ds(bt, rem)], idxct)
            pltpu.sync_copy(x_hbm.at[idxrt], bufrt)
            pltpu.sync_copy(x_hbm.at[idxct], bufct)
            pltpu.sync_copy(bufrt, src_hbm.at[pl.ds(bt, rem)])
            pltpu.sync_copy(bufct, dst_hbm.at[pl.ds(bt, rem)])

    return k(x, row, col)


def _sc_segsum(vals, col, zeros, count_mode=False):
    e, w = vals.shape
    n = zeros.shape[0]
    assert e % NC == 0
    epc = e // NC
    assert epc % NS == 0
    eps = epc // NS
    n_full, rem = divmod(eps, CH)
    assert eps % 8 == 0 and rem % 8 == 0
    rps = (n // NS) & ~7
    rps_last = n - (NS - 1) * rps
    assert rps % 8 == 0 and rps_last % 8 == 0 and rps_last >= rps

    assert n_full % 2 == 0
    npair = n_full // 2

    out_t = jax.ShapeDtypeStruct((n, w), vals.dtype)
    scratch = [
        pltpu.VMEM_SHARED((n, w), vals.dtype),
        pltpu.VMEM((CH,), jnp.int32), pltpu.VMEM((CH,), jnp.int32),
        pltpu.VMEM((CH, w), vals.dtype), pltpu.VMEM((CH, w), vals.dtype),
        pltpu.SemaphoreType.DMA, pltpu.SemaphoreType.DMA,
        pltpu.SemaphoreType.DMA, pltpu.SemaphoreType.DMA,
    ]
    if rem:
        scratch += [pltpu.VMEM((rem,), jnp.int32),
                    pltpu.VMEM((rem, w), vals.dtype)]

    @functools.partial(pl.kernel, out_type=(out_t, out_t), mesh=_mesh(),
                       scratch_types=scratch)
    def k(vals_hbm, col_hbm, zeros_hbm, out0_hbm, out1_hbm, acc_sh,
          idx0, idx1, buf0, buf1, semiv0, semiv1, semsc0, semsc1, *tail):
        idxv, bufv = [idx0, idx1], [buf0, buf1]
        semiv, semsc = [semiv0, semiv1], [semsc0, semsc1]
        core = lax.axis_index("c")
        sid = lax.axis_index("s")

        if count_mode:
            @pl.loop(0, CH)
            def _(r):
                @pl.loop(0, w // LANES)
                def _(cc):
                    buf0[r, pl.ds(cc * LANES, LANES)] = jnp.ones(
                        (LANES,), vals.dtype)

        r0 = sid * rps

        @pl.when(sid < NS - 1)
        def _():
            pltpu.sync_copy(zeros_hbm.at[pl.ds(r0, rps)],
                            acc_sh.at[pl.ds(r0, rps)])

        @pl.when(sid == NS - 1)
        def _():
            pltpu.sync_copy(zeros_hbm.at[pl.ds(r0, rps_last)],
                            acc_sh.at[pl.ds(r0, rps_last)])

        plsc.subcore_barrier()

        base = core * epc + sid * eps

        def iv_cp(p, b):
            cs = [pltpu.make_async_copy(col_hbm.at[pl.ds(b, CH)],
                                        idxv[p], semiv[p])]
            if not count_mode:
                cs.append(pltpu.make_async_copy(vals_hbm.at[pl.ds(b, CH)],
                                                bufv[p], semiv[p]))
            return cs

        def sc_src(p):
            return bufv[0] if count_mode else bufv[p]

        def startall(cs):
            for c in cs:
                c.start()

        def waitall(cs):
            for c in cs:
                c.wait()

        def sc_start(p):
            pltpu.async_copy(sc_src(p), acc_sh.at[idxv[p]], semsc[p],
                             add=True)

        def sc_wait(p):
            pltpu.make_async_copy(sc_src(p), acc_sh.at[idxv[p]],
                                  semsc[p]).wait()

        startall(iv_cp(0, base))
        startall(iv_cp(1, base + CH))

        @pl.loop(0, npair)
        def _(j):
            c0 = base + (2 * j) * CH
            c1 = c0 + CH
            waitall(iv_cp(0, c0))
            sc_start(0)
            waitall(iv_cp(1, c1))
            sc_start(1)

            @pl.when(j < npair - 1)
            def _():
                sc_wait(0)
                startall(iv_cp(0, c0 + 2 * CH))
                sc_wait(1)
                startall(iv_cp(1, c1 + 2 * CH))

        sc_wait(0)
        sc_wait(1)

        if rem:
            idx_t, buf_t = tail
            bt = base + n_full * CH
            pltpu.sync_copy(col_hbm.at[pl.ds(bt, rem)], idx_t)
            if count_mode:
                @pl.loop(0, rem)
                def _(r):
                    @pl.loop(0, w // LANES)
                    def _(cc):
                        buf_t[r, pl.ds(cc * LANES, LANES)] = jnp.ones(
                            (LANES,), vals.dtype)
            else:
                pltpu.sync_copy(vals_hbm.at[pl.ds(bt, rem)], buf_t)
            pltpu.sync_copy(buf_t, acc_sh.at[idx_t], add=True)

        plsc.subcore_barrier()

        for cc, out_hbm in ((0, out0_hbm), (1, out1_hbm)):
            @pl.when((core == cc) & (sid < NS - 1))
            def _():
                pltpu.sync_copy(acc_sh.at[pl.ds(r0, rps)],
                                out_hbm.at[pl.ds(r0, rps)])

            @pl.when((core == cc) & (sid == NS - 1))
            def _():
                pltpu.sync_copy(acc_sh.at[pl.ds(r0, rps_last)],
                                out_hbm.at[pl.ds(r0, rps_last)])

    return k(vals, col, zeros)


def _dot(a, b):
    return jax.lax.dot_general(a.astype(jnp.bfloat16), b.astype(jnp.bfloat16),
                               (((1,), (0,)), ((), ())),
                               preferred_element_type=jnp.float32)


def _tc_edge(src, dst, ea, pe, pn1):
    e, h = ea.shape
    be = 2560
    assert e % be == 0
    grid = (e // be,)

    w1s, w1d, w1e = pe["W1"][:h], pe["W1"][h:2 * h], pe["W1"][2 * h:]
    v1s, v1e = pn1["W1"][:h], pn1["W1"][h:]
    b1 = pe["b1"].reshape(1, h)
    b2 = pe["b2"].reshape(1, h)
    c2 = pn1["b2"].reshape(1, h)
    z = jnp.zeros((h, h), jnp.float32)
    wa = jnp.concatenate([
        jnp.concatenate([w1s, v1s], axis=1),
        jnp.concatenate([w1d, z], axis=1),
        jnp.concatenate([w1e, z], axis=1)], axis=0)
    w2v = jnp.matmul(pe["W2"], v1e, precision=jax.lax.Precision.HIGHEST)
    wb = jnp.concatenate([pe["W2"], w2v], axis=1)
    c1p = (pn1["b1"]
           + jnp.matmul(b2, v1e,
                        precision=jax.lax.Precision.HIGHEST)).reshape(1, h)

    row_spec = pl.BlockSpec((be, h), lambda i: (i, 0))
    wa_spec = pl.BlockSpec((3 * h, 2 * h), lambda i: (0, 0))
    wb_spec = pl.BlockSpec((h, 2 * h), lambda i: (0, 0))
    w_spec = pl.BlockSpec((h, h), lambda i: (0, 0))
    b_spec = pl.BlockSpec((1, h), lambda i: (0, 0))

    def body(src_r, dst_r, ea_r, wa_r, b1_r, wb_r, b2_r, c1p_r, v2_r, c2_r,
             e_out, m_out):
        cat = jnp.concatenate([src_r[...], dst_r[...], ea_r[...]], axis=1)
        a = _dot(cat, wa_r[...])
        hh = jnp.maximum(a[:, :h] + b1_r[...], 0.0)
        gs = a[:, h:]
        bb = _dot(hh, wb_r[...])
        e2 = bb[:, :h] + b2_r[...]
        e_out[...] = e2
        g = jnp.maximum(gs + bb[:, h:] + c1p_r[...], 0.0)
        m_out[...] = _dot(g, v2_r[...]) + c2_r[...]

    sds = jax.ShapeDtypeStruct((e, h), jnp.float32)
    return pl.pallas_call(
        body,
        grid=grid,
        in_specs=[row_spec, row_spec, row_spec,
                  wa_spec, b_spec, wb_spec, b_spec, b_spec, w_spec, b_spec],
        out_specs=[row_spec, row_spec],
        out_shape=(sds, sds),
        compiler_params=pltpu.CompilerParams(
            dimension_semantics=("parallel",)),
    )(src, dst, ea, wa, b1, wb, b2, c1p, pn1["W2"], c2)


def _tc_node(x, s0, s1, c0, c1, pn2):
    n, h = x.shape
    bn = 1000
    assert n % bn == 0
    grid = (n // bn,)

    b1 = pn2["b1"].reshape(1, h)
    b2 = pn2["b2"].reshape(1, h)

    row_spec = pl.BlockSpec((bn, h), lambda i: (i, 0))
    cnt_spec = pl.BlockSpec((bn, h), lambda i: (i, 0))
    w1_spec = pl.BlockSpec((2 * h, h), lambda i: (0, 0))
    w_spec = pl.BlockSpec((h, h), lambda i: (0, 0))
    b_spec = pl.BlockSpec((1, h), lambda i: (0, 0))

    def body(x_r, s0_r, s1_r, c0_r, c1_r, w1_r, b1_r, w2_r, b2_r, out_r):
        cnt = jnp.maximum(c0_r[:, 0:1] + c1_r[:, 0:1], 1.0)
        agg = (s0_r[...] + s1_r[...]) / cnt
        cat = jnp.concatenate([x_r[...], agg], axis=1)
        hh = jnp.maximum(_dot(cat, w1_r[...]) + b1_r[...], 0.0)
        out_r[...] = _dot(hh, w2_r[...]) + b2_r[...]

    return pl.pallas_call(
        body,
        grid=grid,
        in_specs=[row_spec, row_spec, row_spec, cnt_spec, cnt_spec,
                  w1_spec, b_spec, w_spec, b_spec],
        out_specs=pl.BlockSpec((bn, h), lambda i: (i, 0)),
        out_shape=jax.ShapeDtypeStruct((n, h), jnp.float32),
        compiler_params=pltpu.CompilerParams(
            dimension_semantics=("parallel",)),
    )(x, s0, s1, c0, c1, pn2["W1"], b1, pn2["W2"], b2)


def kernel(x, edge_index, edge_attr, params):
    n, h = x.shape
    e = edge_attr.shape[0]
    row = edge_index[0]
    col = edge_index[1]

    zeros_s = jnp.zeros((n, h), jnp.float32)
    c0, c1 = _sc_segsum(edge_attr, col, zeros_s, count_mode=True)

    for p in params:
        src, dst = _sc_gather2(x, row, col)
        e_new, m = _tc_edge(src, dst, edge_attr, p["edge"], p["node1"])
        s0, s1 = _sc_segsum(m, col, zeros_s)
        x = _tc_node(x, s0, s1, c0, c1, p["node2"])
        edge_attr = e_new
    return (x, edge_attr)


# speedup vs baseline: 1.2650x; 1.0283x over previous
import functools

import jax
import jax.numpy as jnp
from jax import lax
from jax.experimental import pallas as pl
from jax.experimental.pallas import tpu as pltpu
from jax.experimental.pallas import tpu_sc as plsc

NC = 2
NS = 16
NW = NC * NS
LANES = 16
CH = 128

def _mesh():
    return plsc.VectorSubcoreMesh(core_axis_name="c", subcore_axis_name="s")


def _sc_gather2(x, row, col, eoff=0, esz=None):
    n, h = x.shape
    esz = row.shape[0] - eoff if esz is None else esz
    assert esz % NW == 0
    epw = esz // NW
    n_full, rem = divmod(epw, CH)
    assert eoff % 8 == 0 and epw % 8 == 0 and rem % 8 == 0
    npair = n_full // 2
    assert npair >= 1

    out_t = jax.ShapeDtypeStruct((esz, h), x.dtype)
    scratch = [
        pltpu.VMEM((CH,), jnp.int32), pltpu.VMEM((CH,), jnp.int32),
        pltpu.VMEM((CH,), jnp.int32), pltpu.VMEM((CH,), jnp.int32),
        pltpu.VMEM((CH, h), x.dtype), pltpu.VMEM((CH, h), x.dtype),
        pltpu.VMEM((CH, h), x.dtype), pltpu.VMEM((CH, h), x.dtype),
        pltpu.SemaphoreType.DMA, pltpu.SemaphoreType.DMA,
        pltpu.SemaphoreType.DMA, pltpu.SemaphoreType.DMA,
        pltpu.SemaphoreType.DMA, pltpu.SemaphoreType.DMA,
    ]
    if rem:
        scratch += [
            pltpu.VMEM((rem,), jnp.int32), pltpu.VMEM((rem,), jnp.int32),
            pltpu.VMEM((rem, h), x.dtype), pltpu.VMEM((rem, h), x.dtype),
        ]

    @functools.partial(pl.kernel, out_type=(out_t, out_t), mesh=_mesh(),
                       scratch_types=scratch)
    def k(x_hbm, row_hbm, col_hbm, src_hbm, dst_hbm,
          idxr0, idxr1, idxc0, idxc1, bufr0, bufr1, bufc0, bufc1,
          semi0, semi1, semg0, semg1, semo0, semo1, *tail):
        idxr, idxc = [idxr0, idxr1], [idxc0, idxc1]
        bufr, bufc = [bufr0, bufr1], [bufc0, bufc1]
        semi, semg, semo = [semi0, semi1], [semg0, semg1], [semo0, semo1]
        wid = lax.axis_index("s") * NC + lax.axis_index("c")
        base_w = wid * epw

        def idx_cp(p, b):
            return (pltpu.make_async_copy(row_hbm.at[pl.ds(eoff + b, CH)],
                                          idxr[p], semi[p]),
                    pltpu.make_async_copy(col_hbm.at[pl.ds(eoff + b, CH)],
                                          idxc[p], semi[p]))

        def gat_cp(p):
            return (pltpu.make_async_copy(x_hbm.at[idxr[p]], bufr[p], semg[p]),
                    pltpu.make_async_copy(x_hbm.at[idxc[p]], bufc[p], semg[p]))

        def out_cp(p, b):
            return (pltpu.make_async_copy(bufr[p], src_hbm.at[pl.ds(b, CH)],
                                          semo[p]),
                    pltpu.make_async_copy(bufc[p], dst_hbm.at[pl.ds(b, CH)],
                                          semo[p]))

        def start2(c):
            c[0].start()
            c[1].start()

        def wait2(c):
            c[0].wait()
            c[1].wait()

        start2(idx_cp(0, base_w))
        start2(idx_cp(1, base_w + CH))
        wait2(idx_cp(0, base_w))
        start2(gat_cp(0))
        wait2(idx_cp(1, base_w + CH))
        start2(gat_cp(1))

        @pl.loop(0, npair)
        def _(j):
            c0 = base_w + (2 * j) * CH
            c1 = c0 + CH
            wait2(gat_cp(0))
            start2(out_cp(0, c0))
            wait2(gat_cp(1))
            start2(out_cp(1, c1))

            @pl.when(j < npair - 1)
            def _():
                start2(idx_cp(0, c0 + 2 * CH))
                start2(idx_cp(1, c1 + 2 * CH))
                wait2(idx_cp(0, c0 + 2 * CH))
                wait2(out_cp(0, c0))
                start2(gat_cp(0))
                wait2(idx_cp(1, c1 + 2 * CH))
                wait2(out_cp(1, c1))
                start2(gat_cp(1))

        blast = base_w + (2 * npair - 2) * CH
        wait2(out_cp(0, blast))
        wait2(out_cp(1, blast + CH))

        if n_full % 2:
            bl = base_w + (n_full - 1) * CH
            pltpu.sync_copy(row_hbm.at[pl.ds(eoff + bl, CH)], idxr0)
            pltpu.sync_copy(col_hbm.at[pl.ds(eoff + bl, CH)], idxc0)
            pltpu.sync_copy(x_hbm.at[idxr0], bufr0)
            pltpu.sync_copy(x_hbm.at[idxc0], bufc0)
            pltpu.sync_copy(bufr0, src_hbm.at[pl.ds(bl, CH)])
            pltpu.sync_copy(bufc0, dst_hbm.at[pl.ds(bl, CH)])

        if rem:
            idxrt, idxct, bufrt, bufct = tail
            bt = base_w + n_full * CH
            pltpu.sync_copy(row_hbm.at[pl.ds(eoff + bt, rem)], idxrt)
            pltpu.sync_copy(col_hbm.at[pl.ds(eoff + bt, rem)], idxct)
            pltpu.sync_copy(x_hbm.at[idxrt], bufrt)
            pltpu.sync_copy(x_hbm.at[idxct], bufct)
            pltpu.sync_copy(bufrt, src_hbm.at[pl.ds(bt, rem)])
            pltpu.sync_copy(bufct, dst_hbm.at[pl.ds(bt, rem)])

    return k(x, row, col)


def _sc_segsum(vals, col, zeros, count_mode=False, eoff=0):
    e, w = vals.shape
    n = zeros.shape[0]
    assert e % NC == 0
    epc = e // NC
    assert epc % NS == 0
    eps = epc // NS
    n_full, rem = divmod(eps, CH)
    assert eoff % 8 == 0 and eps % 8 == 0 and rem % 8 == 0
    rps = (n // NS) & ~7
    rps_last = n - (NS - 1) * rps
    assert rps % 8 == 0 and rps_last % 8 == 0 and rps_last >= rps

    npair = n_full // 2
    assert npair >= 1

    out_t = jax.ShapeDtypeStruct((n, w), vals.dtype)
    scratch = [
        pltpu.VMEM_SHARED((n, w), vals.dtype),
        pltpu.VMEM((CH,), jnp.int32), pltpu.VMEM((CH,), jnp.int32),
        pltpu.VMEM((CH, w), vals.dtype), pltpu.VMEM((CH, w), vals.dtype),
        pltpu.SemaphoreType.DMA, pltpu.SemaphoreType.DMA,
        pltpu.SemaphoreType.DMA, pltpu.SemaphoreType.DMA,
    ]
    if rem:
        scratch += [pltpu.VMEM((rem,), jnp.int32),
                    pltpu.VMEM((rem, w), vals.dtype)]

    @functools.partial(pl.kernel, out_type=(out_t, out_t), mesh=_mesh(),
                       scratch_types=scratch)
    def k(vals_hbm, col_hbm, zeros_hbm, out0_hbm, out1_hbm, acc_sh,
          idx0, idx1, buf0, buf1, semiv0, semiv1, semsc0, semsc1, *tail):
        idxv, bufv = [idx0, idx1], [buf0, buf1]
        semiv, semsc = [semiv0, semiv1], [semsc0, semsc1]
        core = lax.axis_index("c")
        sid = lax.axis_index("s")

        if count_mode:
            @pl.loop(0, CH)
            def _(r):
                @pl.loop(0, w // LANES)
                def _(cc):
                    buf0[r, pl.ds(cc * LANES, LANES)] = jnp.ones(
                        (LANES,), vals.dtype)

        r0 = sid * rps

        @pl.when(sid < NS - 1)
        def _():
            pltpu.sync_copy(zeros_hbm.at[pl.ds(r0, rps)],
                            acc_sh.at[pl.ds(r0, rps)])

        @pl.when(sid == NS - 1)
        def _():
            pltpu.sync_copy(zeros_hbm.at[pl.ds(r0, rps_last)],
                            acc_sh.at[pl.ds(r0, rps_last)])

        plsc.subcore_barrier()

        base = core * epc + sid * eps

        def iv_cp(p, b):
            cs = [pltpu.make_async_copy(col_hbm.at[pl.ds(eoff + b, CH)],
                                        idxv[p], semiv[p])]
            if not count_mode:
                cs.append(pltpu.make_async_copy(vals_hbm.at[pl.ds(b, CH)],
                                                bufv[p], semiv[p]))
            return cs

        def sc_src(p):
            return bufv[0] if count_mode else bufv[p]

        def startall(cs):
            for c in cs:
                c.start()

        def waitall(cs):
            for c in cs:
                c.wait()

        def sc_start(p):
            pltpu.async_copy(sc_src(p), acc_sh.at[idxv[p]], semsc[p],
                             add=True)

        def sc_wait(p):
            pltpu.make_async_copy(sc_src(p), acc_sh.at[idxv[p]],
                                  semsc[p]).wait()

        startall(iv_cp(0, base))
        startall(iv_cp(1, base + CH))

        @pl.loop(0, npair)
        def _(j):
            c0 = base + (2 * j) * CH
            c1 = c0 + CH
            waitall(iv_cp(0, c0))
            sc_start(0)
            waitall(iv_cp(1, c1))
            sc_start(1)

            @pl.when(j < npair - 1)
            def _():
                sc_wait(0)
                startall(iv_cp(0, c0 + 2 * CH))
                sc_wait(1)
                startall(iv_cp(1, c1 + 2 * CH))

        sc_wait(0)
        sc_wait(1)

        if n_full % 2:
            bl = base + (n_full - 1) * CH
            pltpu.sync_copy(col_hbm.at[pl.ds(eoff + bl, CH)], idx0)
            if not count_mode:
                pltpu.sync_copy(vals_hbm.at[pl.ds(bl, CH)], buf0)
            pltpu.sync_copy(buf0, acc_sh.at[idx0], add=True)

        if rem:
            idx_t, buf_t = tail
            bt = base + n_full * CH
            pltpu.sync_copy(col_hbm.at[pl.ds(eoff + bt, rem)], idx_t)
            if count_mode:
                @pl.loop(0, rem)
                def _(r):
                    @pl.loop(0, w // LANES)
                    def _(cc):
                        buf_t[r, pl.ds(cc * LANES, LANES)] = jnp.ones(
                            (LANES,), vals.dtype)
            else:
                pltpu.sync_copy(vals_hbm.at[pl.ds(bt, rem)], buf_t)
            pltpu.sync_copy(buf_t, acc_sh.at[idx_t], add=True)

        plsc.subcore_barrier()

        for cc, out_hbm in ((0, out0_hbm), (1, out1_hbm)):
            @pl.when((core == cc) & (sid < NS - 1))
            def _():
                pltpu.sync_copy(acc_sh.at[pl.ds(r0, rps)],
                                out_hbm.at[pl.ds(r0, rps)])

            @pl.when((core == cc) & (sid == NS - 1))
            def _():
                pltpu.sync_copy(acc_sh.at[pl.ds(r0, rps_last)],
                                out_hbm.at[pl.ds(r0, rps_last)])

    return k(vals, col, zeros)


def _dot(a, b):
    return jax.lax.dot_general(a.astype(jnp.bfloat16), b.astype(jnp.bfloat16),
                               (((1,), (0,)), ((), ())),
                               preferred_element_type=jnp.float32)


def _tc_edge(src, dst, ea, pe, pn1, ea_off=0):
    e, h = src.shape
    be = next(b for b in range(2560, 0, -8) if e % b == 0 and ea_off % b == 0)
    grid = (e // be,)
    ea_blk = ea_off // be

    w1s, w1d, w1e = pe["W1"][:h], pe["W1"][h:2 * h], pe["W1"][2 * h:]
    v1s, v1e = pn1["W1"][:h], pn1["W1"][h:]
    b1 = pe["b1"].reshape(1, h)
    b2 = pe["b2"].reshape(1, h)
    c2 = pn1["b2"].reshape(1, h)
    z = jnp.zeros((h, h), jnp.float32)
    wa = jnp.concatenate([
        jnp.concatenate([w1s, v1s], axis=1),
        jnp.concatenate([w1d, z], axis=1),
        jnp.concatenate([w1e, z], axis=1)], axis=0)
    w2v = jnp.matmul(pe["W2"], v1e, precision=jax.lax.Precision.HIGHEST)
    wb = jnp.concatenate([pe["W2"], w2v], axis=1)
    c1p = (pn1["b1"]
           + jnp.matmul(b2, v1e,
                        precision=jax.lax.Precision.HIGHEST)).reshape(1, h)

    row_spec = pl.BlockSpec((be, h), lambda i: (i, 0))
    ea_spec = pl.BlockSpec((be, h), lambda i: (i + ea_blk, 0))
    wa_spec = pl.BlockSpec((3 * h, 2 * h), lambda i: (0, 0))
    wb_spec = pl.BlockSpec((h, 2 * h), lambda i: (0, 0))
    w_spec = pl.BlockSpec((h, h), lambda i: (0, 0))
    b_spec = pl.BlockSpec((1, h), lambda i: (0, 0))

    def body(src_r, dst_r, ea_r, wa_r, b1_r, wb_r, b2_r, c1p_r, v2_r, c2_r,
             e_out, m_out):
        cat = jnp.concatenate([src_r[...], dst_r[...], ea_r[...]], axis=1)
        a = _dot(cat, wa_r[...])
        hh = jnp.maximum(a[:, :h] + b1_r[...], 0.0)
        gs = a[:, h:]
        bb = _dot(hh, wb_r[...])
        e2 = bb[:, :h] + b2_r[...]
        e_out[...] = e2
        g = jnp.maximum(gs + bb[:, h:] + c1p_r[...], 0.0)
        m_out[...] = _dot(g, v2_r[...]) + c2_r[...]

    sds = jax.ShapeDtypeStruct((e, h), jnp.float32)
    return pl.pallas_call(
        body,
        grid=grid,
        in_specs=[row_spec, row_spec, ea_spec,
                  wa_spec, b_spec, wb_spec, b_spec, b_spec, w_spec, b_spec],
        out_specs=[row_spec, row_spec],
        out_shape=(sds, sds),
        compiler_params=pltpu.CompilerParams(
            dimension_semantics=("parallel",)),
    )(src, dst, ea, wa, b1, wb, b2, c1p, pn1["W2"], c2)


def _tc_node(x, s_parts, c0, c1, pn2):
    n, h = x.shape
    bn = 1000
    assert n % bn == 0
    grid = (n // bn,)
    ns = len(s_parts)

    b1 = pn2["b1"].reshape(1, h)
    b2 = pn2["b2"].reshape(1, h)

    row_spec = pl.BlockSpec((bn, h), lambda i: (i, 0))
    w1_spec = pl.BlockSpec((2 * h, h), lambda i: (0, 0))
    w_spec = pl.BlockSpec((h, h), lambda i: (0, 0))
    b_spec = pl.BlockSpec((1, h), lambda i: (0, 0))

    def body(x_r, *rest):
        s_refs = rest[:ns]
        c0_r, c1_r, w1_r, b1_r, w2_r, b2_r, out_r = rest[ns:]
        cnt = jnp.maximum(c0_r[:, 0:1] + c1_r[:, 0:1], 1.0)
        s = s_refs[0][...]
        for r in s_refs[1:]:
            s = s + r[...]
        agg = s / cnt
        cat = jnp.concatenate([x_r[...], agg], axis=1)
        hh = jnp.maximum(_dot(cat, w1_r[...]) + b1_r[...], 0.0)
        out_r[...] = _dot(hh, w2_r[...]) + b2_r[...]

    return pl.pallas_call(
        body,
        grid=grid,
        in_specs=[row_spec] * (1 + ns) + [row_spec, row_spec,
                                          w1_spec, b_spec, w_spec, b_spec],
        out_specs=pl.BlockSpec((bn, h), lambda i: (i, 0)),
        out_shape=jax.ShapeDtypeStruct((n, h), jnp.float32),
        compiler_params=pltpu.CompilerParams(
            dimension_semantics=("parallel",)),
    )(x, *s_parts, c0, c1, pn2["W1"], b1, pn2["W2"], b2)


def kernel(x, edge_index, edge_attr, params):
    n, h = x.shape
    e = edge_attr.shape[0]
    eh = e // 2
    row = edge_index[0]
    col = edge_index[1]

    zeros_s = jnp.zeros((n, h), jnp.float32)
    c0, c1 = _sc_segsum(edge_attr, col, zeros_s, count_mode=True)

    ea_halves = (edge_attr, edge_attr)
    ea_offs = (0, eh)
    for p in params:
        srcA, dstA = _sc_gather2(x, row, col, eoff=0, esz=eh)
        srcB, dstB = _sc_gather2(x, row, col, eoff=eh, esz=eh)
        eA, mA = _tc_edge(srcA, dstA, ea_halves[0], p["edge"], p["node1"],
                          ea_off=ea_offs[0])
        s0a, s1a = _sc_segsum(mA, col, zeros_s, eoff=0)
        eB, mB = _tc_edge(srcB, dstB, ea_halves[1], p["edge"], p["node1"],
                          ea_off=ea_offs[1])
        s0b, s1b = _sc_segsum(mB, col, zeros_s, eoff=eh)
        x = _tc_node(x, (s0a, s1a, s0b, s1b), c0, c1, p["node2"])
        ea_halves = (eA, eB)
        ea_offs = (0, 0)
    return (x, jnp.concatenate(ea_halves, axis=0))


# speedup vs baseline: 1.3129x; 1.0379x over previous
import functools

import jax
import jax.numpy as jnp
from jax import lax
from jax.experimental import pallas as pl
from jax.experimental.pallas import tpu as pltpu
from jax.experimental.pallas import tpu_sc as plsc

NC = 2
NS = 16
NW = NC * NS
LANES = 16
CH = 128

def _mesh():
    return plsc.VectorSubcoreMesh(core_axis_name="c", subcore_axis_name="s")


def _sc_gather2(x, row, col, eoff=0, esz=None):
    n, h = x.shape
    esz = row.shape[0] - eoff if esz is None else esz
    assert esz % NW == 0
    epw = esz // NW
    n_full, rem = divmod(epw, CH)
    assert eoff % 8 == 0 and epw % 8 == 0 and rem % 8 == 0
    npair = n_full // 2
    assert npair >= 1

    out_t = jax.ShapeDtypeStruct((esz, h), x.dtype)
    scratch = [
        pltpu.VMEM((CH,), jnp.int32), pltpu.VMEM((CH,), jnp.int32),
        pltpu.VMEM((CH,), jnp.int32), pltpu.VMEM((CH,), jnp.int32),
        pltpu.VMEM((CH, h), x.dtype), pltpu.VMEM((CH, h), x.dtype),
        pltpu.VMEM((CH, h), x.dtype), pltpu.VMEM((CH, h), x.dtype),
        pltpu.SemaphoreType.DMA, pltpu.SemaphoreType.DMA,
        pltpu.SemaphoreType.DMA, pltpu.SemaphoreType.DMA,
        pltpu.SemaphoreType.DMA, pltpu.SemaphoreType.DMA,
    ]
    if rem:
        scratch += [
            pltpu.VMEM((rem,), jnp.int32), pltpu.VMEM((rem,), jnp.int32),
            pltpu.VMEM((rem, h), x.dtype), pltpu.VMEM((rem, h), x.dtype),
        ]

    @functools.partial(pl.kernel, out_type=(out_t, out_t), mesh=_mesh(),
                       scratch_types=scratch)
    def k(x_hbm, row_hbm, col_hbm, src_hbm, dst_hbm,
          idxr0, idxr1, idxc0, idxc1, bufr0, bufr1, bufc0, bufc1,
          semi0, semi1, semg0, semg1, semo0, semo1, *tail):
        idxr, idxc = [idxr0, idxr1], [idxc0, idxc1]
        bufr, bufc = [bufr0, bufr1], [bufc0, bufc1]
        semi, semg, semo = [semi0, semi1], [semg0, semg1], [semo0, semo1]
        wid = lax.axis_index("s") * NC + lax.axis_index("c")
        base_w = wid * epw

        def idx_cp(p, b):
            return (pltpu.make_async_copy(row_hbm.at[pl.ds(eoff + b, CH)],
                                          idxr[p], semi[p]),
                    pltpu.make_async_copy(col_hbm.at[pl.ds(eoff + b, CH)],
                                          idxc[p], semi[p]))

        def gat_cp(p):
            return (pltpu.make_async_copy(x_hbm.at[idxr[p]], bufr[p], semg[p]),
                    pltpu.make_async_copy(x_hbm.at[idxc[p]], bufc[p], semg[p]))

        def out_cp(p, b):
            return (pltpu.make_async_copy(bufr[p], src_hbm.at[pl.ds(b, CH)],
                                          semo[p]),
                    pltpu.make_async_copy(bufc[p], dst_hbm.at[pl.ds(b, CH)],
                                          semo[p]))

        def start2(c):
            c[0].start()
            c[1].start()

        def wait2(c):
            c[0].wait()
            c[1].wait()

        start2(idx_cp(0, base_w))
        start2(idx_cp(1, base_w + CH))
        wait2(idx_cp(0, base_w))
        start2(gat_cp(0))
        wait2(idx_cp(1, base_w + CH))
        start2(gat_cp(1))

        @pl.loop(0, npair)
        def _(j):
            c0 = base_w + (2 * j) * CH
            c1 = c0 + CH
            wait2(gat_cp(0))
            start2(out_cp(0, c0))
            wait2(gat_cp(1))
            start2(out_cp(1, c1))

            @pl.when(j < npair - 1)
            def _():
                start2(idx_cp(0, c0 + 2 * CH))
                start2(idx_cp(1, c1 + 2 * CH))
                wait2(idx_cp(0, c0 + 2 * CH))
                wait2(out_cp(0, c0))
                start2(gat_cp(0))
                wait2(idx_cp(1, c1 + 2 * CH))
                wait2(out_cp(1, c1))
                start2(gat_cp(1))

        blast = base_w + (2 * npair - 2) * CH
        wait2(out_cp(0, blast))
        wait2(out_cp(1, blast + CH))

        if n_full % 2:
            bl = base_w + (n_full - 1) * CH
            pltpu.sync_copy(row_hbm.at[pl.ds(eoff + bl, CH)], idxr0)
            pltpu.sync_copy(col_hbm.at[pl.ds(eoff + bl, CH)], idxc0)
            pltpu.sync_copy(x_hbm.at[idxr0], bufr0)
            pltpu.sync_copy(x_hbm.at[idxc0], bufc0)
            pltpu.sync_copy(bufr0, src_hbm.at[pl.ds(bl, CH)])
            pltpu.sync_copy(bufc0, dst_hbm.at[pl.ds(bl, CH)])

        if rem:
            idxrt, idxct, bufrt, bufct = tail
            bt = base_w + n_full * CH
            pltpu.sync_copy(row_hbm.at[pl.ds(eoff + bt, rem)], idxrt)
            pltpu.sync_copy(col_hbm.at[pl.ds(eoff + bt, rem)], idxct)
            pltpu.sync_copy(x_hbm.at[idxrt], bufrt)
            pltpu.sync_copy(x_hbm.at[idxct], bufct)
            pltpu.sync_copy(bufrt, src_hbm.at[pl.ds(bt, rem)])
            pltpu.sync_copy(bufct, dst_hbm.at[pl.ds(bt, rem)])

    return k(x, row, col)


def _sc_segsum(vals, col, zeros, count_mode=False, eoff=0):
    e, w = vals.shape
    n = zeros.shape[0]
    assert e % NC == 0
    epc = e // NC
    assert epc % NS == 0
    eps = epc // NS
    n_full, rem = divmod(eps, CH)
    assert eoff % 8 == 0 and eps % 8 == 0 and rem % 8 == 0
    rps = (n // NS) & ~7
    rps_last = n - (NS - 1) * rps
    assert rps % 8 == 0 and rps_last % 8 == 0 and rps_last >= rps

    npair = n_full // 2
    assert npair >= 1

    out_t = jax.ShapeDtypeStruct((n, w), vals.dtype)
    scratch = [
        pltpu.VMEM_SHARED((n, w), vals.dtype),
        pltpu.VMEM((CH,), jnp.int32), pltpu.VMEM((CH,), jnp.int32),
        pltpu.VMEM((CH, w), vals.dtype), pltpu.VMEM((CH, w), vals.dtype),
        pltpu.SemaphoreType.DMA, pltpu.SemaphoreType.DMA,
        pltpu.SemaphoreType.DMA, pltpu.SemaphoreType.DMA,
    ]
    if rem:
        scratch += [pltpu.VMEM((rem,), jnp.int32),
                    pltpu.VMEM((rem, w), vals.dtype)]

    @functools.partial(pl.kernel, out_type=(out_t, out_t), mesh=_mesh(),
                       scratch_types=scratch)
    def k(vals_hbm, col_hbm, zeros_hbm, out0_hbm, out1_hbm, acc_sh,
          idx0, idx1, buf0, buf1, semiv0, semiv1, semsc0, semsc1, *tail):
        idxv, bufv = [idx0, idx1], [buf0, buf1]
        semiv, semsc = [semiv0, semiv1], [semsc0, semsc1]
        core = lax.axis_index("c")
        sid = lax.axis_index("s")

        if count_mode:
            @pl.loop(0, CH)
            def _(r):
                @pl.loop(0, w // LANES)
                def _(cc):
                    buf0[r, pl.ds(cc * LANES, LANES)] = jnp.ones(
                        (LANES,), vals.dtype)

        r0 = sid * rps

        @pl.when(sid < NS - 1)
        def _():
            pltpu.sync_copy(zeros_hbm.at[pl.ds(r0, rps)],
                            acc_sh.at[pl.ds(r0, rps)])

        @pl.when(sid == NS - 1)
        def _():
            pltpu.sync_copy(zeros_hbm.at[pl.ds(r0, rps_last)],
                            acc_sh.at[pl.ds(r0, rps_last)])

        plsc.subcore_barrier()

        base = core * epc + sid * eps

        def iv_cp(p, b):
            cs = [pltpu.make_async_copy(col_hbm.at[pl.ds(eoff + b, CH)],
                                        idxv[p], semiv[p])]
            if not count_mode:
                cs.append(pltpu.make_async_copy(vals_hbm.at[pl.ds(b, CH)],
                                                bufv[p], semiv[p]))
            return cs

        def sc_src(p):
            return bufv[0] if count_mode else bufv[p]

        def startall(cs):
            for c in cs:
                c.start()

        def waitall(cs):
            for c in cs:
                c.wait()

        def sc_start(p):
            pltpu.async_copy(sc_src(p), acc_sh.at[idxv[p]], semsc[p],
                             add=True)

        def sc_wait(p):
            pltpu.make_async_copy(sc_src(p), acc_sh.at[idxv[p]],
                                  semsc[p]).wait()

        startall(iv_cp(0, base))
        startall(iv_cp(1, base + CH))

        @pl.loop(0, npair)
        def _(j):
            c0 = base + (2 * j) * CH
            c1 = c0 + CH
            waitall(iv_cp(0, c0))
            sc_start(0)
            waitall(iv_cp(1, c1))
            sc_start(1)

            @pl.when(j < npair - 1)
            def _():
                sc_wait(0)
                startall(iv_cp(0, c0 + 2 * CH))
                sc_wait(1)
                startall(iv_cp(1, c1 + 2 * CH))

        sc_wait(0)
        sc_wait(1)

        if n_full % 2:
            bl = base + (n_full - 1) * CH
            pltpu.sync_copy(col_hbm.at[pl.ds(eoff + bl, CH)], idx0)
            if not count_mode:
                pltpu.sync_copy(vals_hbm.at[pl.ds(bl, CH)], buf0)
            pltpu.sync_copy(buf0, acc_sh.at[idx0], add=True)

        if rem:
            idx_t, buf_t = tail
            bt = base + n_full * CH
            pltpu.sync_copy(col_hbm.at[pl.ds(eoff + bt, rem)], idx_t)
            if count_mode:
                @pl.loop(0, rem)
                def _(r):
                    @pl.loop(0, w // LANES)
                    def _(cc):
                        buf_t[r, pl.ds(cc * LANES, LANES)] = jnp.ones(
                            (LANES,), vals.dtype)
            else:
                pltpu.sync_copy(vals_hbm.at[pl.ds(bt, rem)], buf_t)
            pltpu.sync_copy(buf_t, acc_sh.at[idx_t], add=True)

        plsc.subcore_barrier()

        for cc, out_hbm in ((0, out0_hbm), (1, out1_hbm)):
            @pl.when((core == cc) & (sid < NS - 1))
            def _():
                pltpu.sync_copy(acc_sh.at[pl.ds(r0, rps)],
                                out_hbm.at[pl.ds(r0, rps)])

            @pl.when((core == cc) & (sid == NS - 1))
            def _():
                pltpu.sync_copy(acc_sh.at[pl.ds(r0, rps_last)],
                                out_hbm.at[pl.ds(r0, rps_last)])

    return k(vals, col, zeros)


def _dot(a, b):
    return jax.lax.dot_general(a.astype(jnp.bfloat16), b.astype(jnp.bfloat16),
                               (((1,), (0,)), ((), ())),
                               preferred_element_type=jnp.float32)


def _tc_edge(src, dst, ea, pe, pn1, ea_off=0, e_total=None, out_off=0,
             e_alias=None):
    e, h = src.shape
    e_total = e if e_total is None else e_total
    be = next(b for b in range(2560, 0, -8)
              if e % b == 0 and ea_off % b == 0 and e_total % b == 0
              and out_off % b == 0)
    grid = (e // be,)
    ea_blk = ea_off // be
    out_blk = out_off // be

    w1s, w1d, w1e = pe["W1"][:h], pe["W1"][h:2 * h], pe["W1"][2 * h:]
    v1s, v1e = pn1["W1"][:h], pn1["W1"][h:]
    b1 = pe["b1"].reshape(1, h)
    b2 = pe["b2"].reshape(1, h)
    c2 = pn1["b2"].reshape(1, h)
    z = jnp.zeros((h, h), jnp.float32)
    wa = jnp.concatenate([
        jnp.concatenate([w1s, v1s], axis=1),
        jnp.concatenate([w1d, z], axis=1),
        jnp.concatenate([w1e, z], axis=1)], axis=0)
    w2v = jnp.matmul(pe["W2"], v1e, precision=jax.lax.Precision.HIGHEST)
    wb = jnp.concatenate([pe["W2"], w2v], axis=1)
    c1p = (pn1["b1"]
           + jnp.matmul(b2, v1e,
                        precision=jax.lax.Precision.HIGHEST)).reshape(1, h)

    row_spec = pl.BlockSpec((be, h), lambda i: (i, 0))
    ea_spec = pl.BlockSpec((be, h), lambda i: (i + ea_blk, 0))
    wa_spec = pl.BlockSpec((3 * h, 2 * h), lambda i: (0, 0))
    wb_spec = pl.BlockSpec((h, 2 * h), lambda i: (0, 0))
    w_spec = pl.BlockSpec((h, h), lambda i: (0, 0))
    b_spec = pl.BlockSpec((1, h), lambda i: (0, 0))

    def body(src_r, dst_r, ea_r, wa_r, b1_r, wb_r, b2_r, c1p_r, v2_r, c2_r,
             *rest):
        e_out, m_out = rest[-2:]
        cat = jnp.concatenate([src_r[...], dst_r[...], ea_r[...]], axis=1)
        a = _dot(cat, wa_r[...])
        hh = jnp.maximum(a[:, :h] + b1_r[...], 0.0)
        gs = a[:, h:]
        bb = _dot(hh, wb_r[...])
        e2 = bb[:, :h] + b2_r[...]
        e_out[...] = e2
        g = jnp.maximum(gs + bb[:, h:] + c1p_r[...], 0.0)
        m_out[...] = _dot(g, v2_r[...]) + c2_r[...]

    e_spec = pl.BlockSpec((be, h), lambda i: (i + out_blk, 0))
    in_specs = [row_spec, row_spec, ea_spec,
                wa_spec, b_spec, wb_spec, b_spec, b_spec, w_spec, b_spec]
    args = [src, dst, ea, wa, b1, wb, b2, c1p, pn1["W2"], c2]
    aliases = {}
    if e_alias is not None:
        in_specs.append(pl.BlockSpec((8, h), lambda i: (0, 0)))
        args.append(e_alias)
        aliases = {len(args) - 1: 0}
    return pl.pallas_call(
        body,
        grid=grid,
        in_specs=in_specs,
        out_specs=[e_spec, row_spec],
        out_shape=(jax.ShapeDtypeStruct((e_total, h), jnp.float32),
                   jax.ShapeDtypeStruct((e, h), jnp.float32)),
        input_output_aliases=aliases,
        compiler_params=pltpu.CompilerParams(
            dimension_semantics=("parallel",)),
    )(*args)


def _tc_node(x, s_parts, c0, c1, pn2):
    n, h = x.shape
    bn = 1000
    assert n % bn == 0
    grid = (n // bn,)
    ns = len(s_parts)

    b1 = pn2["b1"].reshape(1, h)
    b2 = pn2["b2"].reshape(1, h)

    row_spec = pl.BlockSpec((bn, h), lambda i: (i, 0))
    w1_spec = pl.BlockSpec((2 * h, h), lambda i: (0, 0))
    w_spec = pl.BlockSpec((h, h), lambda i: (0, 0))
    b_spec = pl.BlockSpec((1, h), lambda i: (0, 0))

    def body(x_r, *rest):
        s_refs = rest[:ns]
        c0_r, c1_r, w1_r, b1_r, w2_r, b2_r, out_r = rest[ns:]
        cnt = jnp.maximum(c0_r[:, 0:1] + c1_r[:, 0:1], 1.0)
        s = s_refs[0][...]
        for r in s_refs[1:]:
            s = s + r[...]
        agg = s / cnt
        cat = jnp.concatenate([x_r[...], agg], axis=1)
        hh = jnp.maximum(_dot(cat, w1_r[...]) + b1_r[...], 0.0)
        out_r[...] = _dot(hh, w2_r[...]) + b2_r[...]

    return pl.pallas_call(
        body,
        grid=grid,
        in_specs=[row_spec] * (1 + ns) + [row_spec, row_spec,
                                          w1_spec, b_spec, w_spec, b_spec],
        out_specs=pl.BlockSpec((bn, h), lambda i: (i, 0)),
        out_shape=jax.ShapeDtypeStruct((n, h), jnp.float32),
        compiler_params=pltpu.CompilerParams(
            dimension_semantics=("parallel",)),
    )(x, *s_parts, c0, c1, pn2["W1"], b1, pn2["W2"], b2)


def kernel(x, edge_index, edge_attr, params):
    n, h = x.shape
    e = edge_attr.shape[0]
    eh = e // 2
    row = edge_index[0]
    col = edge_index[1]

    zeros_s = jnp.zeros((n, h), jnp.float32)
    c0, c1 = _sc_segsum(edge_attr, col, zeros_s, count_mode=True)

    ea_full = edge_attr
    for p in params:
        srcA, dstA = _sc_gather2(x, row, col, eoff=0, esz=eh)
        srcB, dstB = _sc_gather2(x, row, col, eoff=eh, esz=eh)
        ea_part, mA = _tc_edge(srcA, dstA, ea_full, p["edge"], p["node1"],
                               ea_off=0, e_total=e, out_off=0)
        s0a, s1a = _sc_segsum(mA, col, zeros_s, eoff=0)
        ea_new, mB = _tc_edge(srcB, dstB, ea_full, p["edge"], p["node1"],
                              ea_off=eh, e_total=e, out_off=eh,
                              e_alias=ea_part)
        s0b, s1b = _sc_segsum(mB, col, zeros_s, eoff=eh)
        x = _tc_node(x, (s0a, s1a, s0b, s1b), c0, c1, p["node2"])
        ea_full = ea_new
    return (x, ea_full)
